# Initial kernel scaffold; baseline (speedup 1.0000x reference)
import jax
import jax.numpy as jnp
from jax.experimental import pallas as pl


def kernel(seq, edge_index, edge_weight, W):
    raise NotImplementedError("write your pallas kernel here")


# speedup vs baseline: 21.8950x; 21.8950x over previous
import functools

import jax
import jax.numpy as jnp
from jax import lax
from jax.experimental import pallas as pl
from jax.experimental.pallas import tpu as pltpu
from jax.experimental.pallas import tpu_sc as plsc

N = 10000
E = 320000
D = 128

NC = 2
NS = 16
NW = NC * NS
CE = 128
CH = 80
EPW = CH * CE
EP = NW * EPW
NP = 10240
RPT = NP // NW
SPT = NP // NS

_mesh = plsc.VectorSubcoreMesh(core_axis_name="c", subcore_axis_name="s")


def _deg_body(row_hbm, ew_hbm, deg_out, row_v, ew_v, zbuf, deg_sh):
    cid = lax.axis_index("c")
    sid = lax.axis_index("s")
    wid = sid * NC + cid

    def _z(i, _):
        zbuf[pl.ds(i * 16, 16)] = jnp.zeros((16,), jnp.float32)
        return 0
    lax.fori_loop(0, SPT // 16, _z, 0)
    pltpu.sync_copy(zbuf, deg_sh.at[pl.ds(sid * SPT, SPT)])
    plsc.subcore_barrier()

    pltpu.sync_copy(row_hbm.at[wid], row_v)
    pltpu.sync_copy(ew_hbm.at[wid], ew_v)

    def _chunk(j, _):
        pltpu.sync_copy(ew_v.at[pl.ds(j * CE, CE)],
                        deg_sh.at[row_v.at[j]], add=True)
        return 0
    lax.fori_loop(0, CH, _chunk, 0)
    plsc.subcore_barrier()

    pltpu.sync_copy(deg_sh.at[pl.ds(sid * SPT, SPT)],
                    deg_out.at[cid, pl.ds(sid * SPT, SPT)])


_deg_kernel = pl.kernel(
    _deg_body,
    out_type=jax.ShapeDtypeStruct((NC, NP), jnp.float32),
    mesh=_mesh,
    scratch_types=[
        pltpu.VMEM((CH, CE), jnp.int32),
        pltpu.VMEM((EPW,), jnp.float32),
        pltpu.VMEM((SPT,), jnp.float32),
        pltpu.VMEM_SHARED((NP,), jnp.float32),
    ],
    compiler_params=pltpu.CompilerParams(needs_layout_passes=False),
)


def _scale_body(deg_ref, seq_ref, o_ref):
    p = deg_ref[...]
    dinv = lax.rsqrt(p[0] + p[1] + 3.0)
    o_ref[...] = seq_ref[...] * dinv[:, None]


def _scale_kernel(deg_parts, seqp):
    br = 1024
    return pl.pallas_call(
        _scale_body,
        grid=(NP // br,),
        in_specs=[
            pl.BlockSpec((NC, br), lambda i: (0, i)),
            pl.BlockSpec((br, D), lambda i: (i, 0)),
        ],
        out_specs=pl.BlockSpec((br, D), lambda i: (i, 0)),
        out_shape=jax.ShapeDtypeStruct((NP, D), jnp.float32),
    )(deg_parts, seqp)


def _spmm_body(s2_hbm, col_hbm, row_hbm, ew_hbm, out_hbm,
               col_v, row_v, ew_v, buf0, out_sh):
    cid = lax.axis_index("c")
    sid = lax.axis_index("s")
    wid = sid * NC + cid

    def _z(i, _):
        buf0[i // 8, pl.ds((i % 8) * 16, 16)] = jnp.zeros((16,), jnp.float32)
        return 0
    lax.fori_loop(0, CE * 8, _z, 0)
    for k in range(SPT // CE):
        pltpu.sync_copy(buf0, out_sh.at[pl.ds(sid * SPT + k * CE, CE)])
    plsc.subcore_barrier()

    pltpu.sync_copy(col_hbm.at[wid], col_v)
    pltpu.sync_copy(row_hbm.at[wid], row_v)
    pltpu.sync_copy(ew_hbm.at[wid], ew_v)

    def _chunk(j, _):
        pltpu.sync_copy(s2_hbm.at[col_v.at[j]], buf0)

        def _edge(e, _):
            w = plsc.load_gather(ew_v, [jnp.full((16,), j * CE + e, jnp.int32)])
            for k in range(D // 16):
                buf0[e, pl.ds(k * 16, 16)] = buf0[e, pl.ds(k * 16, 16)] * w
            return 0
        lax.fori_loop(0, CE, _edge, 0)

        pltpu.sync_copy(buf0, out_sh.at[row_v.at[j]], add=True)
        return 0
    lax.fori_loop(0, CH, _chunk, 0)
    plsc.subcore_barrier()

    for k in range(SPT // CE):
        pltpu.sync_copy(out_sh.at[pl.ds(sid * SPT + k * CE, CE)],
                        out_hbm.at[cid, pl.ds(sid * SPT + k * CE, CE)])


_spmm_kernel = pl.kernel(
    _spmm_body,
    out_type=jax.ShapeDtypeStruct((NC, NP, D), jnp.float32),
    mesh=_mesh,
    scratch_types=[
        pltpu.VMEM((CH, CE), jnp.int32),
        pltpu.VMEM((CH, CE), jnp.int32),
        pltpu.VMEM((EPW,), jnp.float32),
        pltpu.VMEM((CE, D), jnp.float32),
        pltpu.VMEM_SHARED((NP, D), jnp.float32),
    ],
    compiler_params=pltpu.CompilerParams(needs_layout_passes=False),
)


def _final_body(deg_ref, parts_ref, s2_ref, w_ref, o_ref):
    p = deg_ref[...]
    dinv = lax.rsqrt(p[0] + p[1] + 3.0)
    acc = parts_ref[0] + parts_ref[1] + 3.0 * s2_ref[...]
    x = acc * dinv[:, None]
    y = lax.dot_general(x, w_ref[...], (((1,), (1,)), ((), ())),
                        preferred_element_type=jnp.float32)
    o_ref[...] = jnp.maximum(y, 0.0)


def _final_kernel(deg_parts, parts, s2, W):
    br = 1024
    return pl.pallas_call(
        _final_body,
        grid=(NP // br,),
        in_specs=[
            pl.BlockSpec((NC, br), lambda i: (0, i)),
            pl.BlockSpec((NC, br, D), lambda i: (0, i, 0)),
            pl.BlockSpec((br, D), lambda i: (i, 0)),
            pl.BlockSpec((D, D), lambda i: (0, 0)),
        ],
        out_specs=pl.BlockSpec((br, D), lambda i: (i, 0)),
        out_shape=jax.ShapeDtypeStruct((NP, D), jnp.float32),
    )(deg_parts, parts, s2, W)


@jax.jit
def kernel(seq, edge_index, edge_weight, W):
    row = edge_index[0]
    col = edge_index[1]
    pad = EP - E
    pad_idx = (jnp.arange(pad, dtype=jnp.int32) % N)
    rowp = jnp.concatenate([row, pad_idx]).reshape(NW, CH, CE)
    colp = jnp.concatenate([col, pad_idx]).reshape(NW, CH, CE)
    ewp = jnp.concatenate(
        [edge_weight, jnp.zeros((pad,), jnp.float32)]).reshape(NW, EPW)
    seqp = jnp.pad(seq, ((0, NP - N), (0, 0)))

    deg_parts = _deg_kernel(rowp, ewp)
    s2 = _scale_kernel(deg_parts, seqp)
    parts = _spmm_kernel(s2, colp, rowp, ewp)
    outp = _final_kernel(deg_parts, parts, s2, W)
    return outp[:N]


# speedup vs baseline: 30.6933x; 1.4018x over previous
import functools

import jax
import jax.numpy as jnp
from jax import lax
from jax.experimental import pallas as pl
from jax.experimental.pallas import tpu as pltpu
from jax.experimental.pallas import tpu_sc as plsc

N = 10000
E = 320000
D = 128

NC = 2
NS = 16
NW = NC * NS
CE = 128
CH = 80
EPW = CH * CE
EP = NW * EPW
NP = 10240
RPT = NP // NW
SPT = NP // NS

_mesh = plsc.VectorSubcoreMesh(core_axis_name="c", subcore_axis_name="s")


def _deg_body(row_hbm, ew_hbm, deg_out, row_v, ew_v, zbuf, deg_sh):
    cid = lax.axis_index("c")
    sid = lax.axis_index("s")
    wid = sid * NC + cid

    def _z(i, _):
        zbuf[pl.ds(i * 16, 16)] = jnp.zeros((16,), jnp.float32)
        return 0
    lax.fori_loop(0, SPT // 16, _z, 0)
    pltpu.sync_copy(zbuf, deg_sh.at[pl.ds(sid * SPT, SPT)])
    plsc.subcore_barrier()

    pltpu.sync_copy(row_hbm.at[wid], row_v)
    pltpu.sync_copy(ew_hbm.at[wid], ew_v)

    def _chunk(j, _):
        pltpu.sync_copy(ew_v.at[pl.ds(j * CE, CE)],
                        deg_sh.at[row_v.at[j]], add=True)
        return 0
    lax.fori_loop(0, CH, _chunk, 0)
    plsc.subcore_barrier()

    pltpu.sync_copy(deg_sh.at[pl.ds(sid * SPT, SPT)],
                    deg_out.at[cid, pl.ds(sid * SPT, SPT)])


_deg_kernel = pl.kernel(
    _deg_body,
    out_type=jax.ShapeDtypeStruct((NC, NP), jnp.float32),
    mesh=_mesh,
    scratch_types=[
        pltpu.VMEM((CH, CE), jnp.int32),
        pltpu.VMEM((EPW,), jnp.float32),
        pltpu.VMEM((SPT,), jnp.float32),
        pltpu.VMEM_SHARED((NP,), jnp.float32),
    ],
    compiler_params=pltpu.CompilerParams(needs_layout_passes=False),
)


def _scale_body(deg_ref, seq_ref, o_ref):
    p = deg_ref[...]
    dinv = lax.rsqrt(p[0] + p[1] + 3.0)
    o_ref[...] = seq_ref[...] * dinv[:, None]


def _scale_kernel(deg_parts, seqp):
    br = 1024
    return pl.pallas_call(
        _scale_body,
        grid=(NP // br,),
        in_specs=[
            pl.BlockSpec((NC, br), lambda i: (0, i)),
            pl.BlockSpec((br, D), lambda i: (i, 0)),
        ],
        out_specs=pl.BlockSpec((br, D), lambda i: (i, 0)),
        out_shape=jax.ShapeDtypeStruct((NP, D), jnp.float32),
    )(deg_parts, seqp)


def _spmm_body(s2_hbm, col_hbm, row_hbm, ew_hbm, out_hbm,
               colb0, colb1, rowb0, rowb1, ew_v, buf0, buf1, out_sh,
               csem0, csem1, rsem0, rsem1, gsem0, gsem1, ssem0, ssem1):
    cid = lax.axis_index("c")
    sid = lax.axis_index("s")
    wid = sid * NC + cid

    def _z(i, _):
        buf0[i // 8, pl.ds((i % 8) * 16, 16)] = jnp.zeros((16,), jnp.float32)
        return 0
    lax.fori_loop(0, CE * 8, _z, 0)
    for k in range(SPT // CE):
        pltpu.sync_copy(buf0, out_sh.at[pl.ds(sid * SPT + k * CE, CE)])
    plsc.subcore_barrier()

    pltpu.sync_copy(ew_hbm.at[wid], ew_v)

    def _scale(j, buf):
        def _edge(e, _):
            w = plsc.load_gather(ew_v, [jnp.full((16,), j * CE + e, jnp.int32)])
            for k in range(D // 16):
                buf[e, pl.ds(k * 16, 16)] = buf[e, pl.ds(k * 16, 16)] * w
            return 0
        lax.fori_loop(0, CE, _edge, 0)

    def _cstart(j, colb, sem):
        pltpu.async_copy(col_hbm.at[wid, j], colb, sem)

    def _cwait(j, colb, sem):
        pltpu.make_async_copy(col_hbm.at[wid, j], colb, sem).wait()

    def _rstart(j, rowb, sem):
        pltpu.async_copy(row_hbm.at[wid, pl.ds(j, 1)], rowb, sem)

    def _rwait(j, rowb, sem):
        pltpu.make_async_copy(row_hbm.at[wid, pl.ds(j, 1)], rowb, sem).wait()

    def _gstart(colb, buf, sem):
        pltpu.async_copy(s2_hbm.at[colb], buf, sem)

    def _gwait(colb, buf, sem):
        pltpu.make_async_copy(s2_hbm.at[colb], buf, sem).wait()

    def _sstart(rowb, buf, sem):
        pltpu.async_copy(buf, out_sh.at[rowb.at[0]], sem, add=True)

    def _swait(rowb, buf, sem):
        pltpu.make_async_copy(buf, out_sh.at[rowb.at[0]], sem).wait()

    _cstart(0, colb0, csem0)
    _cstart(1, colb1, csem1)
    _rstart(0, rowb0, rsem0)
    _rstart(1, rowb1, rsem1)
    _cwait(0, colb0, csem0)
    _gstart(colb0, buf0, gsem0)
    _cwait(1, colb1, csem1)
    _gstart(colb1, buf1, gsem1)

    def _pair(j2, _):
        j = 2 * j2
        _gwait(colb0, buf0, gsem0)
        _cstart(j + 2, colb0, csem0)
        _scale(j, buf0)
        _rwait(j, rowb0, rsem0)
        _sstart(rowb0, buf0, ssem0)

        _gwait(colb1, buf1, gsem1)
        _cstart(j + 3, colb1, csem1)
        _scale(j + 1, buf1)
        _rwait(j + 1, rowb1, rsem1)
        _sstart(rowb1, buf1, ssem1)

        _swait(rowb0, buf0, ssem0)
        _rstart(j + 2, rowb0, rsem0)
        _cwait(j + 2, colb0, csem0)
        _gstart(colb0, buf0, gsem0)

        _swait(rowb1, buf1, ssem1)
        _rstart(j + 3, rowb1, rsem1)
        _cwait(j + 3, colb1, csem1)
        _gstart(colb1, buf1, gsem1)
        return 0
    lax.fori_loop(0, CH // 2 - 1, _pair, 0)

    j = CH - 2
    _gwait(colb0, buf0, gsem0)
    _scale(j, buf0)
    _rwait(j, rowb0, rsem0)
    _sstart(rowb0, buf0, ssem0)
    _gwait(colb1, buf1, gsem1)
    _scale(j + 1, buf1)
    _rwait(j + 1, rowb1, rsem1)
    _sstart(rowb1, buf1, ssem1)
    _swait(rowb0, buf0, ssem0)
    _swait(rowb1, buf1, ssem1)
    plsc.subcore_barrier()

    for k in range(SPT // CE):
        pltpu.sync_copy(out_sh.at[pl.ds(sid * SPT + k * CE, CE)],
                        out_hbm.at[cid, pl.ds(sid * SPT + k * CE, CE)])


_spmm_kernel = pl.kernel(
    _spmm_body,
    out_type=jax.ShapeDtypeStruct((NC, NP, D), jnp.float32),
    mesh=_mesh,
    scratch_types=[
        pltpu.VMEM((CE,), jnp.int32),
        pltpu.VMEM((CE,), jnp.int32),
        pltpu.VMEM((1, CE), jnp.int32),
        pltpu.VMEM((1, CE), jnp.int32),
        pltpu.VMEM((EPW,), jnp.float32),
        pltpu.VMEM((CE, D), jnp.float32),
        pltpu.VMEM((CE, D), jnp.float32),
        pltpu.VMEM_SHARED((NP, D), jnp.float32),
        pltpu.SemaphoreType.DMA,
        pltpu.SemaphoreType.DMA,
        pltpu.SemaphoreType.DMA,
        pltpu.SemaphoreType.DMA,
        pltpu.SemaphoreType.DMA,
        pltpu.SemaphoreType.DMA,
        pltpu.SemaphoreType.DMA,
        pltpu.SemaphoreType.DMA,
    ],
    compiler_params=pltpu.CompilerParams(needs_layout_passes=False),
)


def _final_body(deg_ref, parts_ref, s2_ref, w_ref, o_ref):
    p = deg_ref[...]
    dinv = lax.rsqrt(p[0] + p[1] + 3.0)
    acc = parts_ref[0] + parts_ref[1] + 3.0 * s2_ref[...]
    x = acc * dinv[:, None]
    y = lax.dot_general(x, w_ref[...], (((1,), (1,)), ((), ())),
                        preferred_element_type=jnp.float32)
    o_ref[...] = jnp.maximum(y, 0.0)


def _final_kernel(deg_parts, parts, s2, W):
    br = 1024
    return pl.pallas_call(
        _final_body,
        grid=(NP // br,),
        in_specs=[
            pl.BlockSpec((NC, br), lambda i: (0, i)),
            pl.BlockSpec((NC, br, D), lambda i: (0, i, 0)),
            pl.BlockSpec((br, D), lambda i: (i, 0)),
            pl.BlockSpec((D, D), lambda i: (0, 0)),
        ],
        out_specs=pl.BlockSpec((br, D), lambda i: (i, 0)),
        out_shape=jax.ShapeDtypeStruct((NP, D), jnp.float32),
    )(deg_parts, parts, s2, W)


@jax.jit
def kernel(seq, edge_index, edge_weight, W):
    row = edge_index[0]
    col = edge_index[1]
    pad = EP - E
    pad_idx = (jnp.arange(pad, dtype=jnp.int32) % N)
    rowp = jnp.concatenate([row, pad_idx]).reshape(NW, CH, CE)
    colp = jnp.concatenate([col, pad_idx]).reshape(NW, CH, CE)
    ewp = jnp.concatenate(
        [edge_weight, jnp.zeros((pad,), jnp.float32)]).reshape(NW, EPW)
    seqp = jnp.pad(seq, ((0, NP - N), (0, 0)))

    deg_parts = _deg_kernel(rowp, ewp)
    s2 = _scale_kernel(deg_parts, seqp)
    parts = _spmm_kernel(s2, colp, rowp, ewp)
    outp = _final_kernel(deg_parts, parts, s2, W)
    return outp[:N]
